# baseline (speedup 1.0000x reference)
import jax
import jax.numpy as jnp
from jax import lax
from jax.experimental import pallas as pl
from jax.experimental.pallas import tpu as pltpu
from jax.experimental.pallas import tpu_sc as plsc

_N = 10000
_NG = 64
_NHID = 128
_DS2S = 3 * _NHID
_STEPS = 10
_NC = 10
_NB = 2000
_NBLK = _N // _NB

_D = 128
_E = 320000
_NTILES = 32
_EPT = _E // _NTILES
_CH = 125
_NCHUNK = _EPT // _CH
_G = 8
_NGRP = _NCHUNK // _G
_NPAD = 10240
_ROWS_PT = _NPAD // 16


def _hp_dot(a, b):
    return lax.dot_general(a, b, (((1,), (0,)), ((), ())),
                           precision=lax.Precision.HIGHEST,
                           preferred_element_type=jnp.float32)


def _seg_dot(seg_mat, v):
    return lax.dot_general(seg_mat, v, (((0,), (0,)), ((), ())),
                           precision=lax.Precision.HIGHEST,
                           preferred_element_type=jnp.float32)


def _s2s_body(batch_ref, x1_ref, x2_ref, x3_ref, wih_ref,
              whh_ref, bih_ref, bhh_ref, wl1_ref, bl1_ref, wl2_ref, bl2_ref,
              wl3_ref, bl3_ref, out_ref,
              h_scr, c_scr, q_scr, qs_scr, e_scr, em_scr, den_scr,
              r1_scr, r2_scr, r3_scr):
    s = pl.program_id(0)
    p = pl.program_id(1)
    n = pl.program_id(2)

    seg = batch_ref[:] == lax.broadcasted_iota(jnp.int32, (_NB, _NG), 1)
    B = seg.astype(jnp.float32)

    @pl.when((p == 0) & (n == 0))
    def _lstm_and_reset():
        @pl.when(s == 0)
        def _zero_state():
            qs_scr[:] = jnp.zeros((_NG, 2 * _DS2S), jnp.float32)
            h_scr[:] = jnp.zeros((_NG, _DS2S), jnp.float32)
            c_scr[:] = jnp.zeros((_NG, _DS2S), jnp.float32)

        q_star = qs_scr[:]
        h = h_scr[:]
        c = c_scr[:]
        gates = (_hp_dot(q_star, wih_ref[:]) + bih_ref[:]
                 + _hp_dot(h, whh_ref[:]) + bhh_ref[:])
        ii = gates[:, :_DS2S]
        ff = gates[:, _DS2S:2 * _DS2S]
        gg = gates[:, 2 * _DS2S:3 * _DS2S]
        oo = gates[:, 3 * _DS2S:]
        c2 = jax.nn.sigmoid(ff) * c + jax.nn.sigmoid(ii) * jnp.tanh(gg)
        h2 = jax.nn.sigmoid(oo) * jnp.tanh(c2)
        h_scr[:] = h2
        c_scr[:] = c2
        q_scr[:] = h2
        em_scr[:] = jnp.full((1, _NG), -1e30, jnp.float32)
        den_scr[:] = jnp.zeros((1, _NG), jnp.float32)
        r1_scr[:] = jnp.zeros((_NG, _NHID), jnp.float32)
        r2_scr[:] = jnp.zeros((_NG, _NHID), jnp.float32)
        r3_scr[:] = jnp.zeros((_NG, _NHID), jnp.float32)

    @pl.when(p == 0)
    def _e_pass():
        q = q_scr[:]
        Bq = _hp_dot(B, q)
        e = (jnp.sum(x1_ref[:] * Bq[:, :_NHID], axis=1, keepdims=True)
             + jnp.sum(x2_ref[:] * Bq[:, _NHID:2 * _NHID], axis=1,
                       keepdims=True)
             + jnp.sum(x3_ref[:] * Bq[:, 2 * _NHID:], axis=1, keepdims=True))
        e_scr[pl.ds(n * _NB, _NB), :] = e
        blkmax = jnp.max(jnp.where(seg, e, -1e30), axis=0, keepdims=True)
        em_scr[:] = jnp.maximum(em_scr[:], blkmax)

    @pl.when(p == 1)
    def _den_pass():
        e = e_scr[pl.ds(n * _NB, _NB), :]
        em_node = jnp.sum(B * em_scr[:], axis=1, keepdims=True)
        ee = jnp.exp(e - em_node)
        den_scr[:] = den_scr[:] + jnp.sum(jnp.where(seg, ee, 0.0), axis=0,
                                          keepdims=True)

    @pl.when(p == 2)
    def _r_pass():
        e = e_scr[pl.ds(n * _NB, _NB), :]
        em_node = jnp.sum(B * em_scr[:], axis=1, keepdims=True)
        den_node = jnp.sum(B * den_scr[:], axis=1, keepdims=True)
        a = jnp.exp(e - em_node) / den_node
        r1_scr[:] = r1_scr[:] + _seg_dot(B, a * x1_ref[:])
        r2_scr[:] = r2_scr[:] + _seg_dot(B, a * x2_ref[:])
        r3_scr[:] = r3_scr[:] + _seg_dot(B, a * x3_ref[:])

        @pl.when(n == _NBLK - 1)
        def _finish_step():
            q_star = jnp.concatenate(
                [q_scr[:], r1_scr[:], r2_scr[:], r3_scr[:]], axis=1)
            qs_scr[:] = q_star

            @pl.when(s == _STEPS - 1)
            def _head():
                hdn = jax.nn.relu(_hp_dot(q_star, wl1_ref[:]) + bl1_ref[:])
                hdn = jax.nn.relu(_hp_dot(hdn, wl2_ref[:]) + bl2_ref[:])
                logits = _hp_dot(hdn, wl3_ref[:]) + bl3_ref[:]
                m = jnp.max(logits, axis=1, keepdims=True)
                lse = m + jnp.log(jnp.sum(jnp.exp(logits - m), axis=1,
                                          keepdims=True))
                out_ref[:] = logits - lse


def _set2set_head(batch, x1, x2, x3, W_ih, W_hh, b_ih, b_hh, Wl1, bl1, Wl2,
                  bl2, Wl3, bl3):
    xmap = lambda s, p, n: (jnp.where(p == 1, _NBLK - 1, n), 0)
    full = lambda s, p, n: (0, 0)
    return pl.pallas_call(
        _s2s_body,
        grid=(_STEPS, 3, _NBLK),
        in_specs=[
            pl.BlockSpec((_NB, 1), lambda s, p, n: (n, 0)),
            pl.BlockSpec((_NB, _NHID), xmap),
            pl.BlockSpec((_NB, _NHID), xmap),
            pl.BlockSpec((_NB, _NHID), xmap),
            pl.BlockSpec((2 * _DS2S, 4 * _DS2S), full),
            pl.BlockSpec((_DS2S, 4 * _DS2S), full),
            pl.BlockSpec((1, 4 * _DS2S), full),
            pl.BlockSpec((1, 4 * _DS2S), full),
            pl.BlockSpec((2 * _DS2S, _NHID), full),
            pl.BlockSpec((1, _NHID), full),
            pl.BlockSpec((_NHID, _NHID // 2), full),
            pl.BlockSpec((1, _NHID // 2), full),
            pl.BlockSpec((_NHID // 2, _NC), full),
            pl.BlockSpec((1, _NC), full),
        ],
        out_specs=pl.BlockSpec((_NG, _NC), full),
        out_shape=jax.ShapeDtypeStruct((_NG, _NC), jnp.float32),
        scratch_shapes=[
            pltpu.VMEM((_NG, _DS2S), jnp.float32),
            pltpu.VMEM((_NG, _DS2S), jnp.float32),
            pltpu.VMEM((_NG, _DS2S), jnp.float32),
            pltpu.VMEM((_NG, 2 * _DS2S), jnp.float32),
            pltpu.VMEM((_N, 1), jnp.float32),
            pltpu.VMEM((1, _NG), jnp.float32),
            pltpu.VMEM((1, _NG), jnp.float32),
            pltpu.VMEM((_NG, _NHID), jnp.float32),
            pltpu.VMEM((_NG, _NHID), jnp.float32),
            pltpu.VMEM((_NG, _NHID), jnp.float32),
        ],
    )(batch.reshape(_N, 1), x1, x2, x3, W_ih, W_hh,
      b_ih.reshape(1, -1), b_hh.reshape(1, -1), Wl1, bl1.reshape(1, -1),
      Wl2, bl2.reshape(1, -1), Wl3, bl3.reshape(1, -1))


def _aggr_body(x_hbm, srcb_hbm, dstb_hbm, zeros_hbm, out_hbm,
               s0, d0, s1, d1, r0, r1, acc_sh, sem0, sem1, semi0, semi1):
    cid = lax.axis_index("c")
    sid = lax.axis_index("s")
    wid = sid * 2 + cid
    rows = (r0, r1)
    sems = (sem0, sem1)
    sidx = (s0, s1)
    didx = (d0, d1)
    semi = (semi0, semi1)
    pltpu.sync_copy(zeros_hbm, acc_sh.at[pl.ds(sid * _ROWS_PT, _ROWS_PT)])
    plsc.subcore_barrier()

    base = wid * _NCHUNK
    pltpu.async_copy(srcb_hbm.at[pl.ds(base, _G)], s0, semi0)
    pltpu.async_copy(dstb_hbm.at[pl.ds(base, _G)], d0, semi0)

    def group(g, par):
        sv = sidx[par]
        dv = didx[par]
        pltpu.make_async_copy(srcb_hbm.at[pl.ds(base, _G)], sv,
                              semi[par]).wait()
        pltpu.make_async_copy(dstb_hbm.at[pl.ds(base, _G)], dv,
                              semi[par]).wait()
        pltpu.async_copy(x_hbm.at[sv.at[0]], rows[0], sems[0])
        pltpu.async_copy(x_hbm.at[sv.at[1]], rows[1], sems[1])

        @pl.when(g + 1 < _NGRP)
        def _prefetch_idx():
            nbase = base + (g + 1) * _G
            pltpu.async_copy(srcb_hbm.at[pl.ds(nbase, _G)], sidx[1 - par],
                             semi[1 - par])
            pltpu.async_copy(dstb_hbm.at[pl.ds(nbase, _G)], didx[1 - par],
                             semi[1 - par])

        for k in range(_G):
            b = k % 2
            pltpu.make_async_copy(x_hbm.at[sv.at[0]], rows[b],
                                  sems[b]).wait()
            pltpu.sync_copy(rows[b], acc_sh.at[dv.at[k]], add=True)
            if k + 2 < _G:
                pltpu.async_copy(x_hbm.at[sv.at[k + 2]], rows[b], sems[b])

    def super_body(u, carry):
        group(2 * u, 0)
        group(2 * u + 1, 1)
        return carry

    lax.fori_loop(0, _NGRP // 2, super_body, 0)
    plsc.subcore_barrier()
    pltpu.sync_copy(
        acc_sh.at[pl.ds(sid * _ROWS_PT, _ROWS_PT)],
        out_hbm.at[pl.ds(cid * _NPAD + sid * _ROWS_PT, _ROWS_PT)])


def _edge_aggregate(h, src_r, dst_r, zeros):
    mesh = plsc.VectorSubcoreMesh(core_axis_name="c", subcore_axis_name="s")
    run = pl.kernel(
        _aggr_body,
        mesh=mesh,
        out_type=jax.ShapeDtypeStruct((2 * _NPAD, _D), jnp.float32),
        scratch_types=[
            pltpu.VMEM((_G, _CH), jnp.int32),
            pltpu.VMEM((_G, _CH), jnp.int32),
            pltpu.VMEM((_G, _CH), jnp.int32),
            pltpu.VMEM((_G, _CH), jnp.int32),
            pltpu.VMEM((_CH, _D), jnp.float32),
            pltpu.VMEM((_CH, _D), jnp.float32),
            pltpu.VMEM_SHARED((_NPAD, _D), jnp.float32),
            pltpu.SemaphoreType.DMA,
            pltpu.SemaphoreType.DMA,
            pltpu.SemaphoreType.DMA,
            pltpu.SemaphoreType.DMA,
        ],
    )
    return run(h, src_r, dst_r, zeros)


def _conv_body(p0_ref, p1_ref, h_ref, wrel_ref, wroot_ref, b_ref, out_ref):
    aggr = p0_ref[:] + p1_ref[:]
    out_ref[:] = jnp.maximum(
        _hp_dot(aggr, wrel_ref[:]) + b_ref[:]
        + _hp_dot(h_ref[:], wroot_ref[:]), 0.0)


def _conv_matmul(p0, p1, h, W_rel, W_root, b):
    full = lambda n: (0, 0)
    blk = lambda n: (n, 0)
    return pl.pallas_call(
        _conv_body,
        grid=(_NBLK,),
        in_specs=[
            pl.BlockSpec((_NB, _D), blk),
            pl.BlockSpec((_NB, _D), blk),
            pl.BlockSpec((_NB, _D), blk),
            pl.BlockSpec((_D, _NHID), full),
            pl.BlockSpec((_D, _NHID), full),
            pl.BlockSpec((1, _NHID), full),
        ],
        out_specs=pl.BlockSpec((_NB, _NHID), blk),
        out_shape=jax.ShapeDtypeStruct((_N, _NHID), jnp.float32),
    )(p0, p1, h, W_rel, W_root, b.reshape(1, -1))


def kernel(x, edge_index, batch, W1_rel, b1, W1_root, W2_rel, b2, W2_root,
           W3_rel, b3, W3_root, W_ih, W_hh, b_ih, b_hh, Wl1, bl1, Wl2, bl2,
           Wl3, bl3):
    order = jnp.argsort(edge_index[1], stable=True)
    src_r = edge_index[0][order].reshape(_E // _CH, _CH)
    dst_r = edge_index[1][order].reshape(_E // _CH, _CH)
    zeros = jnp.zeros((_ROWS_PT, _D), jnp.float32)

    def conv(h, W_rel, b, W_root):
        parts = _edge_aggregate(h, src_r, dst_r, zeros)
        return _conv_matmul(parts[:_N], parts[_NPAD:_NPAD + _N], h,
                            W_rel, W_root, b)

    x1 = conv(x, W1_rel, b1, W1_root)
    x2 = conv(x1, W2_rel, b2, W2_root)
    x3 = conv(x2, W3_rel, b3, W3_root)
    return _set2set_head(batch, x1, x2, x3, W_ih, W_hh, b_ih, b_hh,
                         Wl1, bl1, Wl2, bl2, Wl3, bl3)


# speedup vs baseline: 1.4648x; 1.4648x over previous
import jax
import jax.numpy as jnp
from jax import lax
from jax.experimental import pallas as pl
from jax.experimental.pallas import tpu as pltpu
from jax.experimental.pallas import tpu_sc as plsc

_N = 10000
_NG = 64
_NHID = 128
_DS2S = 3 * _NHID
_STEPS = 10
_NC = 10
_NB = 2000
_NBLK = _N // _NB

_D = 128
_E = 320000
_NTILES = 32
_EPT = _E // _NTILES
_CH = 125
_NCHUNK = _EPT // _CH
_G = 8
_NGRP = _NCHUNK // _G
_NPAD = 10240
_ROWS_PT = _NPAD // 16


def _hp_dot(a, b):
    return lax.dot_general(a, b, (((1,), (0,)), ((), ())),
                           precision=lax.Precision.HIGHEST,
                           preferred_element_type=jnp.float32)


def _seg_dot(seg_mat, v):
    return lax.dot_general(seg_mat, v, (((0,), (0,)), ((), ())),
                           precision=lax.Precision.HIGHEST,
                           preferred_element_type=jnp.float32)


def _s2s_body(batch_ref, x1_ref, x2_ref, x3_ref, wih_ref,
              whh_ref, bih_ref, bhh_ref, wl1_ref, bl1_ref, wl2_ref, bl2_ref,
              wl3_ref, bl3_ref, out_ref,
              h_scr, c_scr, q_scr, qs_scr, e_scr, em_scr, den_scr,
              r1_scr, r2_scr, r3_scr):
    s = pl.program_id(0)
    p = pl.program_id(1)
    n = pl.program_id(2)

    seg = batch_ref[:] == lax.broadcasted_iota(jnp.int32, (_NB, _NG), 1)
    B = seg.astype(jnp.float32)

    @pl.when((p == 0) & (n == 0))
    def _lstm_and_reset():
        @pl.when(s == 0)
        def _zero_state():
            qs_scr[:] = jnp.zeros((_NG, 2 * _DS2S), jnp.float32)
            h_scr[:] = jnp.zeros((_NG, _DS2S), jnp.float32)
            c_scr[:] = jnp.zeros((_NG, _DS2S), jnp.float32)

        q_star = qs_scr[:]
        h = h_scr[:]
        c = c_scr[:]
        gates = (_hp_dot(q_star, wih_ref[:]) + bih_ref[:]
                 + _hp_dot(h, whh_ref[:]) + bhh_ref[:])
        ii = gates[:, :_DS2S]
        ff = gates[:, _DS2S:2 * _DS2S]
        gg = gates[:, 2 * _DS2S:3 * _DS2S]
        oo = gates[:, 3 * _DS2S:]
        c2 = jax.nn.sigmoid(ff) * c + jax.nn.sigmoid(ii) * jnp.tanh(gg)
        h2 = jax.nn.sigmoid(oo) * jnp.tanh(c2)
        h_scr[:] = h2
        c_scr[:] = c2
        q_scr[:] = h2
        em_scr[:] = jnp.full((1, _NG), -1e30, jnp.float32)
        den_scr[:] = jnp.zeros((1, _NG), jnp.float32)
        r1_scr[:] = jnp.zeros((_NG, _NHID), jnp.float32)
        r2_scr[:] = jnp.zeros((_NG, _NHID), jnp.float32)
        r3_scr[:] = jnp.zeros((_NG, _NHID), jnp.float32)

    @pl.when(p == 0)
    def _e_pass():
        q = q_scr[:]
        Bq = _hp_dot(B, q)
        e = (jnp.sum(x1_ref[:] * Bq[:, :_NHID], axis=1, keepdims=True)
             + jnp.sum(x2_ref[:] * Bq[:, _NHID:2 * _NHID], axis=1,
                       keepdims=True)
             + jnp.sum(x3_ref[:] * Bq[:, 2 * _NHID:], axis=1, keepdims=True))
        e_scr[pl.ds(n * _NB, _NB), :] = e
        blkmax = jnp.max(jnp.where(seg, e, -1e30), axis=0, keepdims=True)
        em_scr[:] = jnp.maximum(em_scr[:], blkmax)

    @pl.when(p == 1)
    def _den_pass():
        e = e_scr[pl.ds(n * _NB, _NB), :]
        em_node = jnp.sum(B * em_scr[:], axis=1, keepdims=True)
        ee = jnp.exp(e - em_node)
        den_scr[:] = den_scr[:] + jnp.sum(jnp.where(seg, ee, 0.0), axis=0,
                                          keepdims=True)

    @pl.when(p == 2)
    def _r_pass():
        e = e_scr[pl.ds(n * _NB, _NB), :]
        em_node = jnp.sum(B * em_scr[:], axis=1, keepdims=True)
        den_node = jnp.sum(B * den_scr[:], axis=1, keepdims=True)
        a = jnp.exp(e - em_node) / den_node
        r1_scr[:] = r1_scr[:] + _seg_dot(B, a * x1_ref[:])
        r2_scr[:] = r2_scr[:] + _seg_dot(B, a * x2_ref[:])
        r3_scr[:] = r3_scr[:] + _seg_dot(B, a * x3_ref[:])

        @pl.when(n == _NBLK - 1)
        def _finish_step():
            q_star = jnp.concatenate(
                [q_scr[:], r1_scr[:], r2_scr[:], r3_scr[:]], axis=1)
            qs_scr[:] = q_star

            @pl.when(s == _STEPS - 1)
            def _head():
                hdn = jax.nn.relu(_hp_dot(q_star, wl1_ref[:]) + bl1_ref[:])
                hdn = jax.nn.relu(_hp_dot(hdn, wl2_ref[:]) + bl2_ref[:])
                logits = _hp_dot(hdn, wl3_ref[:]) + bl3_ref[:]
                m = jnp.max(logits, axis=1, keepdims=True)
                lse = m + jnp.log(jnp.sum(jnp.exp(logits - m), axis=1,
                                          keepdims=True))
                out_ref[:] = logits - lse


def _set2set_head(batch, x1, x2, x3, W_ih, W_hh, b_ih, b_hh, Wl1, bl1, Wl2,
                  bl2, Wl3, bl3):
    xmap = lambda s, p, n: (jnp.where(p == 1, _NBLK - 1, n), 0)
    full = lambda s, p, n: (0, 0)
    return pl.pallas_call(
        _s2s_body,
        grid=(_STEPS, 3, _NBLK),
        in_specs=[
            pl.BlockSpec((_NB, 1), lambda s, p, n: (n, 0)),
            pl.BlockSpec((_NB, _NHID), xmap),
            pl.BlockSpec((_NB, _NHID), xmap),
            pl.BlockSpec((_NB, _NHID), xmap),
            pl.BlockSpec((2 * _DS2S, 4 * _DS2S), full),
            pl.BlockSpec((_DS2S, 4 * _DS2S), full),
            pl.BlockSpec((1, 4 * _DS2S), full),
            pl.BlockSpec((1, 4 * _DS2S), full),
            pl.BlockSpec((2 * _DS2S, _NHID), full),
            pl.BlockSpec((1, _NHID), full),
            pl.BlockSpec((_NHID, _NHID // 2), full),
            pl.BlockSpec((1, _NHID // 2), full),
            pl.BlockSpec((_NHID // 2, _NC), full),
            pl.BlockSpec((1, _NC), full),
        ],
        out_specs=pl.BlockSpec((_NG, _NC), full),
        out_shape=jax.ShapeDtypeStruct((_NG, _NC), jnp.float32),
        scratch_shapes=[
            pltpu.VMEM((_NG, _DS2S), jnp.float32),
            pltpu.VMEM((_NG, _DS2S), jnp.float32),
            pltpu.VMEM((_NG, _DS2S), jnp.float32),
            pltpu.VMEM((_NG, 2 * _DS2S), jnp.float32),
            pltpu.VMEM((_N, 1), jnp.float32),
            pltpu.VMEM((1, _NG), jnp.float32),
            pltpu.VMEM((1, _NG), jnp.float32),
            pltpu.VMEM((_NG, _NHID), jnp.float32),
            pltpu.VMEM((_NG, _NHID), jnp.float32),
            pltpu.VMEM((_NG, _NHID), jnp.float32),
        ],
    )(batch.reshape(_N, 1), x1, x2, x3, W_ih, W_hh,
      b_ih.reshape(1, -1), b_hh.reshape(1, -1), Wl1, bl1.reshape(1, -1),
      Wl2, bl2.reshape(1, -1), Wl3, bl3.reshape(1, -1))


def _aggr_body(x_hbm, srcb_hbm, dstb_hbm, zeros_hbm, out_hbm,
               s0, d0, s1, d1, r0, r1, acc_sh, sem0, sem1, semi0, semi1):
    cid = lax.axis_index("c")
    sid = lax.axis_index("s")
    wid = sid * 2 + cid
    rows = (r0, r1)
    sems = (sem0, sem1)
    sidx = (s0, s1)
    didx = (d0, d1)
    semi = (semi0, semi1)
    pltpu.sync_copy(zeros_hbm, acc_sh.at[pl.ds(sid * _ROWS_PT, _ROWS_PT)])
    plsc.subcore_barrier()

    base = wid * _NCHUNK
    pltpu.async_copy(srcb_hbm.at[pl.ds(base, _G)], s0, semi0)
    pltpu.async_copy(dstb_hbm.at[pl.ds(base, _G)], d0, semi0)

    def group(g, par):
        sv = sidx[par]
        dv = didx[par]
        pltpu.make_async_copy(srcb_hbm.at[pl.ds(base, _G)], sv,
                              semi[par]).wait()
        pltpu.make_async_copy(dstb_hbm.at[pl.ds(base, _G)], dv,
                              semi[par]).wait()
        pltpu.async_copy(x_hbm.at[sv.at[0]], rows[0], sems[0])
        pltpu.async_copy(x_hbm.at[sv.at[1]], rows[1], sems[1])

        @pl.when(g + 1 < _NGRP)
        def _prefetch_idx():
            nbase = base + (g + 1) * _G
            pltpu.async_copy(srcb_hbm.at[pl.ds(nbase, _G)], sidx[1 - par],
                             semi[1 - par])
            pltpu.async_copy(dstb_hbm.at[pl.ds(nbase, _G)], didx[1 - par],
                             semi[1 - par])

        for k in range(_G):
            b = k % 2
            pltpu.make_async_copy(x_hbm.at[sv.at[0]], rows[b],
                                  sems[b]).wait()
            pltpu.sync_copy(rows[b], acc_sh.at[dv.at[k]], add=True)
            if k + 2 < _G:
                pltpu.async_copy(x_hbm.at[sv.at[k + 2]], rows[b], sems[b])

    def super_body(u, carry):
        group(2 * u, 0)
        group(2 * u + 1, 1)
        return carry

    lax.fori_loop(0, _NGRP // 2, super_body, 0)
    plsc.subcore_barrier()
    pltpu.sync_copy(
        acc_sh.at[pl.ds(sid * _ROWS_PT, _ROWS_PT)],
        out_hbm.at[pl.ds(cid * _NPAD + sid * _ROWS_PT, _ROWS_PT)])


def _edge_aggregate(h, src_r, dst_r, zeros):
    mesh = plsc.VectorSubcoreMesh(core_axis_name="c", subcore_axis_name="s")
    run = pl.kernel(
        _aggr_body,
        mesh=mesh,
        out_type=jax.ShapeDtypeStruct((2 * _NPAD, _D), jnp.float32),
        scratch_types=[
            pltpu.VMEM((_G, _CH), jnp.int32),
            pltpu.VMEM((_G, _CH), jnp.int32),
            pltpu.VMEM((_G, _CH), jnp.int32),
            pltpu.VMEM((_G, _CH), jnp.int32),
            pltpu.VMEM((_CH, _D), jnp.float32),
            pltpu.VMEM((_CH, _D), jnp.float32),
            pltpu.VMEM_SHARED((_NPAD, _D), jnp.float32),
            pltpu.SemaphoreType.DMA,
            pltpu.SemaphoreType.DMA,
            pltpu.SemaphoreType.DMA,
            pltpu.SemaphoreType.DMA,
        ],
    )
    return run(h, src_r, dst_r, zeros)


def _conv_body(p0_ref, p1_ref, h_ref, wrel_ref, wroot_ref, b_ref, out_ref):
    aggr = p0_ref[:] + p1_ref[:]
    out_ref[:] = jnp.maximum(
        _hp_dot(aggr, wrel_ref[:]) + b_ref[:]
        + _hp_dot(h_ref[:], wroot_ref[:]), 0.0)


def _conv_matmul(p0, p1, h, W_rel, W_root, b):
    full = lambda n: (0, 0)
    blk = lambda n: (n, 0)
    return pl.pallas_call(
        _conv_body,
        grid=(_NBLK,),
        in_specs=[
            pl.BlockSpec((_NB, _D), blk),
            pl.BlockSpec((_NB, _D), blk),
            pl.BlockSpec((_NB, _D), blk),
            pl.BlockSpec((_D, _NHID), full),
            pl.BlockSpec((_D, _NHID), full),
            pl.BlockSpec((1, _NHID), full),
        ],
        out_specs=pl.BlockSpec((_NB, _NHID), blk),
        out_shape=jax.ShapeDtypeStruct((_N, _NHID), jnp.float32),
    )(p0, p1, h, W_rel, W_root, b.reshape(1, -1))


def kernel(x, edge_index, batch, W1_rel, b1, W1_root, W2_rel, b2, W2_root,
           W3_rel, b3, W3_root, W_ih, W_hh, b_ih, b_hh, Wl1, bl1, Wl2, bl2,
           Wl3, bl3):
    src_r = edge_index[0].reshape(_E // _CH, _CH)
    dst_r = edge_index[1].reshape(_E // _CH, _CH)
    zeros = jnp.zeros((_ROWS_PT, _D), jnp.float32)

    def conv(h, W_rel, b, W_root):
        parts = _edge_aggregate(h, src_r, dst_r, zeros)
        return _conv_matmul(parts[:_N], parts[_NPAD:_NPAD + _N], h,
                            W_rel, W_root, b)

    x1 = conv(x, W1_rel, b1, W1_root)
    x2 = conv(x1, W2_rel, b2, W2_root)
    x3 = conv(x2, W3_rel, b3, W3_root)
    return _set2set_head(batch, x1, x2, x3, W_ih, W_hh, b_ih, b_hh,
                         Wl1, bl1, Wl2, bl2, Wl3, bl3)
